# baseline (speedup 1.0000x reference)
import functools

import jax
import jax.numpy as jnp
from jax import lax
from jax.experimental import pallas as pl
from jax.experimental.pallas import tpu as pltpu
from jax.experimental.pallas import tpu_sc as plsc

B = 16
MAX_LEN = 2048
TOTAL = 16384
VOCAB = 1000
DIM = 128

NC, NS = 2, 16
NW = NC * NS
RPW = B * MAX_LEN // NW
CHUNK = 128
NCH = RPW // CHUNK
TOKBUF = RPW + 16
ZROW = VOCAB


def _body(tok_hbm, cu_hbm, tab_hbm, emb_hbm, len_hbm,
          tok_v, idx_v, rows0, rows1, cu_v, len_v, sem0, sem1):
    c = lax.axis_index("c")
    s = lax.axis_index("s")
    w = s * NC + c
    b = w // 2
    t0 = (w % 2) * RPW

    pltpu.sync_copy(cu_hbm, cu_v)
    bounds = cu_v[pl.ds(b, 16)]
    start = bounds[0]
    seq_len = bounds[1] - start

    base = start + t0
    base_al = jnp.minimum((base // 8) * 8, TOTAL - TOKBUF)
    off = base - base_al
    pltpu.sync_copy(tok_hbm.at[pl.ds(base_al, TOKBUF)], tok_v)

    iota = lax.iota(jnp.int32, 16)
    for j in range(RPW // 16):
        t_vec = iota + (t0 + j * 16)
        valid = t_vec < seq_len
        src = jnp.minimum(off + (j * 16) + iota, TOKBUF - 1)
        tok16 = plsc.load_gather(tok_v, [src])
        idx_v[j // 8, pl.ds((j % 8) * 16, 16)] = jnp.where(valid, tok16, ZROW)

    rows = (rows0, rows1)
    sems = (sem0, sem1)
    copies = [
        pltpu.make_async_copy(tab_hbm.at[idx_v.at[ci]], rows[ci % 2], sems[ci % 2])
        for ci in range(NCH)
    ]
    copies[0].start()
    for ci in range(NCH):
        copies[ci].wait()
        if ci + 1 < NCH:
            copies[ci + 1].start()
        pltpu.sync_copy(rows[ci % 2], emb_hbm.at[pl.ds(w * RPW + ci * CHUNK, CHUNK)])

    @pl.when(w == 0)
    def _():
        lo = plsc.load_gather(cu_v, [iota])
        hi = plsc.load_gather(cu_v, [iota + 1])
        len_v[...] = hi - lo
        pltpu.sync_copy(len_v, len_hbm)


@jax.jit
def kernel(tokens, cu_seqlens, table):
    tokens = tokens.astype(jnp.int32)
    cu = cu_seqlens.astype(jnp.int32)
    cu_pad = jnp.concatenate([cu, jnp.zeros((32 - (B + 1),), jnp.int32)])
    tab_ext = jnp.concatenate(
        [table.astype(jnp.float32), jnp.zeros((8, DIM), jnp.float32)])

    run = pl.kernel(
        _body,
        out_type=(
            jax.ShapeDtypeStruct((B * MAX_LEN, DIM), jnp.float32),
            jax.ShapeDtypeStruct((B,), jnp.int32),
        ),
        mesh=plsc.VectorSubcoreMesh(core_axis_name="c", subcore_axis_name="s"),
        compiler_params=pltpu.CompilerParams(needs_layout_passes=False),
        scratch_types=(
            pltpu.VMEM((TOKBUF,), jnp.int32),
            pltpu.VMEM((NCH, CHUNK), jnp.int32),
            pltpu.VMEM((CHUNK, DIM), jnp.float32),
            pltpu.VMEM((CHUNK, DIM), jnp.float32),
            pltpu.VMEM((32,), jnp.int32),
            pltpu.VMEM((B,), jnp.int32),
            pltpu.SemaphoreType.DMA,
            pltpu.SemaphoreType.DMA,
        ),
    )
    emb_flat, lengths = run(tokens, cu_pad, tab_ext)
    return emb_flat.reshape(B, MAX_LEN, DIM), lengths


# speedup vs baseline: 21.0168x; 21.0168x over previous
import functools

import jax
import jax.numpy as jnp
from jax import lax
from jax.experimental import pallas as pl
from jax.experimental.pallas import tpu as pltpu
from jax.experimental.pallas import tpu_sc as plsc

B = 16
MAX_LEN = 2048
TOTAL = 16384
VOCAB = 1000
DIM = 128

NC, NS = 2, 16
NW = NC * NS
RPW = B * MAX_LEN // NW
CHUNK = 128
NCH = RPW // CHUNK
TOKBUF = RPW + 16
ZROW = VOCAB


def _body(tok_hbm, cu_hbm, tab_hbm, emb_hbm, len_hbm,
          tok_v, idx_v, rows0, rows1, cu_v, len_v, tab_sh, sem0, sem1):
    c = lax.axis_index("c")
    s = lax.axis_index("s")
    w = s * NC + c
    b = w // 2
    t0 = (w % 2) * RPW

    @pl.when(s == 0)
    def _():
        pltpu.sync_copy(tab_hbm, tab_sh)

    pltpu.sync_copy(cu_hbm, cu_v)
    bounds = cu_v[pl.ds(b, 16)]
    start = bounds[0]
    seq_len = bounds[1] - start

    base = start + t0
    base_al = jnp.minimum((base // 8) * 8, TOTAL - TOKBUF)
    off = base - base_al
    pltpu.sync_copy(tok_hbm.at[pl.ds(base_al, TOKBUF)], tok_v)

    iota = lax.iota(jnp.int32, 16)
    for j in range(RPW // 16):
        t_vec = iota + (t0 + j * 16)
        valid = t_vec < seq_len
        src = jnp.minimum(off + (j * 16) + iota, TOKBUF - 1)
        tok16 = plsc.load_gather(tok_v, [src])
        idx_v[j // 8, pl.ds((j % 8) * 16, 16)] = jnp.where(valid, tok16, ZROW)

    plsc.subcore_barrier()

    rows = (rows0, rows1)
    sems = (sem0, sem1)
    copies = [
        pltpu.make_async_copy(tab_sh.at[idx_v.at[ci]], rows[ci % 2], sems[ci % 2])
        for ci in range(NCH)
    ]
    copies[0].start()
    for ci in range(NCH):
        copies[ci].wait()
        if ci + 1 < NCH:
            copies[ci + 1].start()
        pltpu.sync_copy(rows[ci % 2], emb_hbm.at[pl.ds(w * RPW + ci * CHUNK, CHUNK)])

    @pl.when(w == 0)
    def _():
        lo = plsc.load_gather(cu_v, [iota])
        hi = plsc.load_gather(cu_v, [iota + 1])
        len_v[...] = hi - lo
        pltpu.sync_copy(len_v, len_hbm)


@jax.jit
def kernel(tokens, cu_seqlens, table):
    tokens = tokens.astype(jnp.int32)
    cu = cu_seqlens.astype(jnp.int32)
    cu_pad = jnp.concatenate([cu, jnp.zeros((32 - (B + 1),), jnp.int32)])
    tab_ext = jnp.concatenate(
        [table.astype(jnp.float32), jnp.zeros((8, DIM), jnp.float32)])

    run = pl.kernel(
        _body,
        out_type=(
            jax.ShapeDtypeStruct((B * MAX_LEN, DIM), jnp.float32),
            jax.ShapeDtypeStruct((B,), jnp.int32),
        ),
        mesh=plsc.VectorSubcoreMesh(core_axis_name="c", subcore_axis_name="s"),
        compiler_params=pltpu.CompilerParams(needs_layout_passes=False),
        scratch_types=(
            pltpu.VMEM((TOKBUF,), jnp.int32),
            pltpu.VMEM((NCH, CHUNK), jnp.int32),
            pltpu.VMEM((CHUNK, DIM), jnp.float32),
            pltpu.VMEM((CHUNK, DIM), jnp.float32),
            pltpu.VMEM((32,), jnp.int32),
            pltpu.VMEM((B,), jnp.int32),
            pltpu.VMEM_SHARED((VOCAB + 8, DIM), jnp.float32),
            pltpu.SemaphoreType.DMA,
            pltpu.SemaphoreType.DMA,
        ),
    )
    emb_flat, lengths = run(tokens, cu_pad, tab_ext)
    return emb_flat.reshape(B, MAX_LEN, DIM), lengths


# speedup vs baseline: 23.1206x; 1.1001x over previous
import jax
import jax.numpy as jnp
from jax import lax
from jax.experimental import pallas as pl
from jax.experimental.pallas import tpu as pltpu
from jax.experimental.pallas import tpu_sc as plsc

B = 16
MAX_LEN = 2048
TOTAL = 16384
VOCAB = 1000
DIM = 128

NC, NS = 2, 16
NW = NC * NS
RPW = B * MAX_LEN // NW
CHUNK = 128
NCH = RPW // CHUNK
TOKBUF = RPW + 16
ZROW = VOCAB


def _body(tok_hbm, cu_hbm, tab_hbm, emb_hbm, len_hbm,
          tok_v, idx_v, rows0, rows1, zbuf, zidx, cu_v, len_v, tab_sh,
          sem_in0, sem_in1, sem_out, sem_z):
    c = lax.axis_index("c")
    s = lax.axis_index("s")
    w = s * NC + c
    b = w // 2
    t0 = (w % 2) * RPW

    iota = lax.iota(jnp.int32, 16)

    zero16 = jnp.zeros((16,), jnp.float32)
    for j in range(8 * DIM // 16):
        zbuf[j // 8, pl.ds((j % 8) * 16, 16)] = zero16

    @pl.when(s == 0)
    def _():
        pltpu.sync_copy(tab_hbm, tab_sh.at[pl.ds(0, VOCAB)])
        pltpu.sync_copy(zbuf.at[pl.ds(0, 8)], tab_sh.at[pl.ds(VOCAB, 8)])

    pltpu.sync_copy(cu_hbm, cu_v.at[pl.ds(0, B + 1)])
    bounds = cu_v[pl.ds(b, 16)]
    start = bounds[0]
    seq_len = bounds[1] - start

    valid = [t0 + ci * CHUNK < seq_len for ci in range(NCH)]

    base = start + t0
    base_al = jnp.minimum((base // 8) * 8, TOTAL - TOKBUF)
    off = base - base_al

    @pl.when(valid[0])
    def _():
        pltpu.sync_copy(tok_hbm.at[pl.ds(base_al, TOKBUF)], tok_v)

    for g in range(NCH):
        @pl.when(valid[g])
        def _(g=g):
            for jj in range(CHUNK // 16):
                j = g * (CHUNK // 16) + jj
                t_vec = iota + (t0 + j * 16)
                in_seg = t_vec < seq_len
                src = jnp.minimum(off + (j * 16) + iota, TOKBUF - 1)
                tok16 = plsc.load_gather(tok_v, [src])
                idx_v[g, pl.ds(jj * 16, 16)] = jnp.where(in_seg, tok16, ZROW)

    for j in range(CHUNK // 16):
        zidx[pl.ds(j * 16, 16)] = jnp.full((16,), ZROW, jnp.int32)

    plsc.subcore_barrier()

    zcp = pltpu.make_async_copy(tab_sh.at[zidx], zbuf, sem_z)
    zcp.start()

    rows = (rows0, rows1)
    sems = (sem_in0, sem_in1)
    ins = [
        pltpu.make_async_copy(tab_sh.at[idx_v.at[ci]], rows[ci % 2], sems[ci % 2])
        for ci in range(NCH)
    ]
    outs_r = [
        pltpu.make_async_copy(rows[ci % 2],
                              emb_hbm.at[pl.ds(w * RPW + ci * CHUNK, CHUNK)],
                              sem_out)
        for ci in range(NCH)
    ]
    outs_z = [
        pltpu.make_async_copy(zbuf,
                              emb_hbm.at[pl.ds(w * RPW + ci * CHUNK, CHUNK)],
                              sem_out)
        for ci in range(NCH)
    ]

    @pl.when(valid[0])
    def _():
        ins[0].start()

    zcp.wait()

    for ci in range(NCH):
        @pl.when(valid[ci])
        def _(ci=ci):
            ins[ci].wait()
        if ci >= 1:
            outs_r[ci - 1].wait()
        if ci + 1 < NCH:
            @pl.when(valid[ci + 1])
            def _(ci=ci):
                ins[ci + 1].start()

        @pl.when(valid[ci])
        def _(ci=ci):
            outs_r[ci].start()

        @pl.when(jnp.logical_not(valid[ci]))
        def _(ci=ci):
            outs_z[ci].start()

    outs_r[NCH - 1].wait()

    @pl.when(w == 0)
    def _():
        lo = plsc.load_gather(cu_v, [iota])
        hi = plsc.load_gather(cu_v, [iota + 1])
        len_v[...] = hi - lo
        pltpu.sync_copy(len_v, len_hbm)


@jax.jit
def kernel(tokens, cu_seqlens, table):
    tokens = tokens.astype(jnp.int32)
    cu = cu_seqlens.astype(jnp.int32)
    tab = table.astype(jnp.float32)

    run = pl.kernel(
        _body,
        out_type=(
            jax.ShapeDtypeStruct((B * MAX_LEN, DIM), jnp.float32),
            jax.ShapeDtypeStruct((B,), jnp.int32),
        ),
        mesh=plsc.VectorSubcoreMesh(core_axis_name="c", subcore_axis_name="s"),
        compiler_params=pltpu.CompilerParams(needs_layout_passes=False),
        scratch_types=(
            pltpu.VMEM((TOKBUF,), jnp.int32),
            pltpu.VMEM((NCH, CHUNK), jnp.int32),
            pltpu.VMEM((CHUNK, DIM), jnp.float32),
            pltpu.VMEM((CHUNK, DIM), jnp.float32),
            pltpu.VMEM((CHUNK, DIM), jnp.float32),
            pltpu.VMEM((CHUNK,), jnp.int32),
            pltpu.VMEM((32,), jnp.int32),
            pltpu.VMEM((B,), jnp.int32),
            pltpu.VMEM_SHARED((VOCAB + 8, DIM), jnp.float32),
            pltpu.SemaphoreType.DMA,
            pltpu.SemaphoreType.DMA,
            pltpu.SemaphoreType.DMA,
            pltpu.SemaphoreType.DMA,
        ),
    )
    emb_flat, lengths = run(tokens, cu, tab)
    return emb_flat.reshape(B, MAX_LEN, DIM), lengths


# speedup vs baseline: 24.0915x; 1.0420x over previous
import jax
import jax.numpy as jnp
from jax import lax
from jax.experimental import pallas as pl
from jax.experimental.pallas import tpu as pltpu
from jax.experimental.pallas import tpu_sc as plsc

B = 16
MAX_LEN = 2048
TOTAL = 16384
VOCAB = 1000
DIM = 128

NC, NS = 2, 16
NW = NC * NS
RPW = B * MAX_LEN // NW
CHUNK = 128
NCH = RPW // CHUNK
TOKBUF = RPW + 16
ZROW = VOCAB


def _body(tok_hbm, cu_hbm, tab_hbm, emb_hbm, len_hbm,
          tok_v, idx_v, rows0, rows1, zbuf, zidx, cu_v, len_v, tab_sh,
          sem_in0, sem_in1, sem_out, sem_z, sem_t):
    c = lax.axis_index("c")
    s = lax.axis_index("s")
    w = s * NC + c
    b = w // 2
    t0 = (w % 2) * RPW

    iota = lax.iota(jnp.int32, 16)

    zero16 = jnp.zeros((16,), jnp.float32)
    for j in range(8 * DIM // 16):
        zbuf[j // 8, pl.ds((j % 8) * 16, 16)] = zero16

    tabcp = pltpu.make_async_copy(tab_hbm, tab_sh.at[pl.ds(0, VOCAB)], sem_t)

    @pl.when(s == 0)
    def _():
        tabcp.start()

    pltpu.sync_copy(cu_hbm, cu_v.at[pl.ds(0, B + 1)])
    bounds = cu_v[pl.ds(b, 16)]
    start = bounds[0]
    seq_len = bounds[1] - start

    valid = [t0 + ci * CHUNK < seq_len for ci in range(NCH)]

    base = start + t0
    base_al = jnp.minimum((base // 8) * 8, TOTAL - TOKBUF)
    off = base - base_al

    @pl.when(valid[0])
    def _():
        pltpu.sync_copy(tok_hbm.at[pl.ds(base_al, TOKBUF)], tok_v)

    for g in range(NCH):
        @pl.when(valid[g])
        def _(g=g):
            for jj in range(CHUNK // 16):
                j = g * (CHUNK // 16) + jj
                t_vec = iota + (t0 + j * 16)
                in_seg = t_vec < seq_len
                src = jnp.minimum(off + (j * 16) + iota, TOKBUF - 1)
                tok16 = plsc.load_gather(tok_v, [src])
                idx_v[g, pl.ds(jj * 16, 16)] = jnp.where(in_seg, tok16, ZROW)

    zrow16 = ZROW + (iota % 8)
    for j in range(CHUNK // 16):
        zidx[pl.ds(j * 16, 16)] = zrow16

    @pl.when(s == 0)
    def _():
        tabcp.wait()
        pltpu.sync_copy(zbuf.at[pl.ds(0, 8)], tab_sh.at[pl.ds(VOCAB, 8)])

    plsc.subcore_barrier()

    rows = (rows0, rows1)
    sems = (sem_in0, sem_in1)
    ins = [
        pltpu.make_async_copy(tab_sh.at[idx_v.at[ci]], rows[ci % 2], sems[ci % 2])
        for ci in range(NCH)
    ]
    outs_r = [
        pltpu.make_async_copy(rows[ci % 2],
                              emb_hbm.at[pl.ds(w * RPW + ci * CHUNK, CHUNK)],
                              sem_out)
        for ci in range(NCH)
    ]
    outs_z = [
        pltpu.make_async_copy(zbuf,
                              emb_hbm.at[pl.ds(w * RPW + ci * CHUNK, CHUNK)],
                              sem_out)
        for ci in range(NCH)
    ]

    @pl.when(valid[0])
    def _():
        ins[0].start()

    zcp = pltpu.make_async_copy(tab_sh.at[zidx], zbuf, sem_z)
    zcp.start()
    zcp.wait()

    for ci in range(NCH):
        @pl.when(valid[ci])
        def _(ci=ci):
            ins[ci].wait()
        if ci >= 1:
            outs_r[ci - 1].wait()
        if ci + 1 < NCH:
            @pl.when(valid[ci + 1])
            def _(ci=ci):
                ins[ci + 1].start()

        @pl.when(valid[ci])
        def _(ci=ci):
            outs_r[ci].start()

        @pl.when(jnp.logical_not(valid[ci]))
        def _(ci=ci):
            outs_z[ci].start()

    outs_r[NCH - 1].wait()

    @pl.when(w == 0)
    def _():
        lo = plsc.load_gather(cu_v, [iota])
        hi = plsc.load_gather(cu_v, [iota + 1])
        len_v[...] = hi - lo
        pltpu.sync_copy(len_v, len_hbm)


@jax.jit
def kernel(tokens, cu_seqlens, table):
    tokens = tokens.astype(jnp.int32)
    cu = cu_seqlens.astype(jnp.int32)
    tab = table.astype(jnp.float32)

    run = pl.kernel(
        _body,
        out_type=(
            jax.ShapeDtypeStruct((B * MAX_LEN, DIM), jnp.float32),
            jax.ShapeDtypeStruct((B,), jnp.int32),
        ),
        mesh=plsc.VectorSubcoreMesh(core_axis_name="c", subcore_axis_name="s"),
        compiler_params=pltpu.CompilerParams(needs_layout_passes=False),
        scratch_types=(
            pltpu.VMEM((TOKBUF,), jnp.int32),
            pltpu.VMEM((NCH, CHUNK), jnp.int32),
            pltpu.VMEM((CHUNK, DIM), jnp.float32),
            pltpu.VMEM((CHUNK, DIM), jnp.float32),
            pltpu.VMEM((CHUNK, DIM), jnp.float32),
            pltpu.VMEM((CHUNK,), jnp.int32),
            pltpu.VMEM((32,), jnp.int32),
            pltpu.VMEM((B,), jnp.int32),
            pltpu.VMEM_SHARED((VOCAB + 8, DIM), jnp.float32),
            pltpu.SemaphoreType.DMA,
            pltpu.SemaphoreType.DMA,
            pltpu.SemaphoreType.DMA,
            pltpu.SemaphoreType.DMA,
            pltpu.SemaphoreType.DMA,
        ),
    )
    emb_flat, lengths = run(tokens, cu, tab)
    return emb_flat.reshape(B, MAX_LEN, DIM), lengths


# speedup vs baseline: 24.2492x; 1.0065x over previous
import jax
import jax.numpy as jnp
from jax import lax
from jax.experimental import pallas as pl
from jax.experimental.pallas import tpu as pltpu
from jax.experimental.pallas import tpu_sc as plsc

B = 16
MAX_LEN = 2048
TOTAL = 16384
VOCAB = 1000
DIM = 128

NC, NS = 2, 16
NW = NC * NS
RPW = B * MAX_LEN // NW
CHUNK = 128
NCH = RPW // CHUNK
TOKBUF = RPW + 16
ZROW = VOCAB


def _body(tok_hbm, cu_hbm, tab_hbm, emb_hbm, len_hbm,
          tok_v, idx_v, rows0, rows1, zbuf, zidx, cu_v, len_v, tab_sh,
          sem_in0, sem_in1, sem_out, sem_z, sem_t):
    c = lax.axis_index("c")
    s = lax.axis_index("s")
    w = s * NC + c
    b = w // 2
    t0 = (w % 2) * RPW

    iota = lax.iota(jnp.int32, 16)

    zero16 = jnp.zeros((16,), jnp.float32)
    for j in range(8 * DIM // 16):
        zbuf[j // 8, pl.ds((j % 8) * 16, 16)] = zero16

    tabcp = pltpu.make_async_copy(tab_hbm, tab_sh.at[pl.ds(0, VOCAB)], sem_t)

    @pl.when(s == 0)
    def _():
        tabcp.start()

    pltpu.sync_copy(cu_hbm, cu_v.at[pl.ds(0, B + 1)])
    bounds = cu_v[pl.ds(b, 16)]
    start = bounds[0]
    seq_len = bounds[1] - start

    valid = [t0 + ci * CHUNK < seq_len for ci in range(NCH)]

    base = start + t0
    base_al = jnp.minimum((base // 8) * 8, TOTAL - TOKBUF)
    off = base - base_al

    @pl.when(valid[0])
    def _():
        pltpu.sync_copy(tok_hbm.at[pl.ds(base_al, TOKBUF)], tok_v)

    def build_group(g):
        @pl.when(valid[g])
        def _():
            for jj in range(CHUNK // 16):
                j = g * (CHUNK // 16) + jj
                t_vec = iota + (t0 + j * 16)
                in_seg = t_vec < seq_len
                src = jnp.minimum(off + (j * 16) + iota, TOKBUF - 1)
                tok16 = plsc.load_gather(tok_v, [src])
                idx_v[g, pl.ds(jj * 16, 16)] = jnp.where(in_seg, tok16, ZROW)

    build_group(0)

    zrow16 = ZROW + (iota % 8)
    for j in range(CHUNK // 16):
        zidx[pl.ds(j * 16, 16)] = zrow16

    @pl.when(s == 0)
    def _():
        tabcp.wait()
        pltpu.sync_copy(zbuf.at[pl.ds(0, 8)], tab_sh.at[pl.ds(VOCAB, 8)])

    plsc.subcore_barrier()

    rows = (rows0, rows1)
    sems = (sem_in0, sem_in1)
    ins = [
        pltpu.make_async_copy(tab_sh.at[idx_v.at[ci]], rows[ci % 2], sems[ci % 2])
        for ci in range(NCH)
    ]
    outs_r = [
        pltpu.make_async_copy(rows[ci % 2],
                              emb_hbm.at[pl.ds(w * RPW + ci * CHUNK, CHUNK)],
                              sem_out)
        for ci in range(NCH)
    ]
    outs_z = [
        pltpu.make_async_copy(zbuf,
                              emb_hbm.at[pl.ds(w * RPW + ci * CHUNK, CHUNK)],
                              sem_out)
        for ci in range(NCH)
    ]

    @pl.when(valid[0])
    def _():
        ins[0].start()

    zcp = pltpu.make_async_copy(tab_sh.at[zidx], zbuf, sem_z)
    zcp.start()

    for g in range(1, NCH):
        build_group(g)

    for ci in range(NCH):
        first_invalid = (jnp.logical_not(valid[ci]) if ci == 0
                         else jnp.logical_and(valid[ci - 1],
                                              jnp.logical_not(valid[ci])))

        @pl.when(first_invalid)
        def _():
            zcp.wait()

        @pl.when(valid[ci])
        def _(ci=ci):
            ins[ci].wait()
        if ci >= 1:
            outs_r[ci - 1].wait()
        if ci + 1 < NCH:
            @pl.when(valid[ci + 1])
            def _(ci=ci):
                ins[ci + 1].start()

        @pl.when(valid[ci])
        def _(ci=ci):
            outs_r[ci].start()

        @pl.when(jnp.logical_not(valid[ci]))
        def _(ci=ci):
            outs_z[ci].start()

    @pl.when(valid[NCH - 1])
    def _():
        zcp.wait()

    outs_r[NCH - 1].wait()

    @pl.when(w == 0)
    def _():
        lo = plsc.load_gather(cu_v, [iota])
        hi = plsc.load_gather(cu_v, [iota + 1])
        len_v[...] = hi - lo
        pltpu.sync_copy(len_v, len_hbm)


@jax.jit
def kernel(tokens, cu_seqlens, table):
    tokens = tokens.astype(jnp.int32)
    cu = cu_seqlens.astype(jnp.int32)
    tab = table.astype(jnp.float32)

    run = pl.kernel(
        _body,
        out_type=(
            jax.ShapeDtypeStruct((B * MAX_LEN, DIM), jnp.float32),
            jax.ShapeDtypeStruct((B,), jnp.int32),
        ),
        mesh=plsc.VectorSubcoreMesh(core_axis_name="c", subcore_axis_name="s"),
        compiler_params=pltpu.CompilerParams(needs_layout_passes=False),
        scratch_types=(
            pltpu.VMEM((TOKBUF,), jnp.int32),
            pltpu.VMEM((NCH, CHUNK), jnp.int32),
            pltpu.VMEM((CHUNK, DIM), jnp.float32),
            pltpu.VMEM((CHUNK, DIM), jnp.float32),
            pltpu.VMEM((CHUNK, DIM), jnp.float32),
            pltpu.VMEM((CHUNK,), jnp.int32),
            pltpu.VMEM((32,), jnp.int32),
            pltpu.VMEM((B,), jnp.int32),
            pltpu.VMEM_SHARED((VOCAB + 8, DIM), jnp.float32),
            pltpu.SemaphoreType.DMA,
            pltpu.SemaphoreType.DMA,
            pltpu.SemaphoreType.DMA,
            pltpu.SemaphoreType.DMA,
            pltpu.SemaphoreType.DMA,
        ),
    )
    emb_flat, lengths = run(tokens, cu, tab)
    return emb_flat.reshape(B, MAX_LEN, DIM), lengths
